# baseline (speedup 1.0000x reference)
import functools

import jax
import jax.numpy as jnp
from jax import lax
from jax.experimental import pallas as pl
from jax.experimental.pallas import tpu as pltpu, tpu_sc as plsc

_B = 16384
_T = 1000
_NS = 16
_L = 16
_BPW = _B // _NS


def _ddpm_body(t_hbm, beta_hbm, alpha_hbm, beta_out, alpha_out,
               idx_v, beta_v, alpha_v, bout_v, aout_v, sem_in, sem_out):
    wid = lax.axis_index("s")
    base = wid * _BPW
    cp_i = pltpu.async_copy(t_hbm.at[pl.ds(base, _BPW)], idx_v, sem_in)
    cp_b = pltpu.async_copy(beta_hbm, beta_v, sem_in)
    cp_a = pltpu.async_copy(alpha_hbm, alpha_v, sem_in)
    cp_i.wait()
    cp_b.wait()
    for i in range(_BPW // _L):
        idx = idx_v[pl.ds(i * _L, _L)]
        bout_v[pl.ds(i * _L, _L)] = plsc.load_gather(beta_v, [idx])
    cp_ob = pltpu.async_copy(bout_v, beta_out.at[pl.ds(base, _BPW)], sem_out)
    cp_a.wait()
    for i in range(_BPW // _L):
        idx = idx_v[pl.ds(i * _L, _L)]
        aout_v[pl.ds(i * _L, _L)] = plsc.load_gather(alpha_v, [idx])
    cp_oa = pltpu.async_copy(aout_v, alpha_out.at[pl.ds(base, _BPW)], sem_out)
    cp_ob.wait()
    cp_oa.wait()


_ddpm = functools.partial(
    pl.kernel,
    mesh=plsc.VectorSubcoreMesh(core_axis_name="c", subcore_axis_name="s",
                                num_cores=1),
    out_type=(
        jax.ShapeDtypeStruct((_B,), jnp.float32),
        jax.ShapeDtypeStruct((_B,), jnp.float32),
    ),
    scratch_types=[
        pltpu.VMEM((_BPW,), jnp.int32),
        pltpu.VMEM((_T,), jnp.float32),
        pltpu.VMEM((_T,), jnp.float32),
        pltpu.VMEM((_BPW,), jnp.float32),
        pltpu.VMEM((_BPW,), jnp.float32),
        pltpu.SemaphoreType.DMA,
        pltpu.SemaphoreType.DMA,
    ],
    compiler_params=pltpu.CompilerParams(needs_layout_passes=False),
)(_ddpm_body)


@jax.jit
def kernel(t, beta, alpha):
    beta_t, alpha_t = _ddpm(t, beta, alpha)
    return beta_t, alpha_t


# speedup vs baseline: 1.0062x; 1.0062x over previous
import functools

import jax
import jax.numpy as jnp
from jax import lax
from jax.experimental import pallas as pl
from jax.experimental.pallas import tpu as pltpu, tpu_sc as plsc

_B = 16384
_T = 1000
_NS = 16
_L = 16
_BPW = _B // _NS


def _ddpm_body(t_hbm, beta_hbm, alpha_hbm, beta_out, alpha_out,
               idx_v, beta_v, alpha_v, bout_v, aout_v, sem_in, sem_out):
    wid = lax.axis_index("s")
    base = wid * _BPW
    cp_i = pltpu.async_copy(t_hbm.at[pl.ds(base, _BPW)], idx_v, sem_in)
    cp_b = pltpu.async_copy(beta_hbm, beta_v, sem_in)
    cp_a = pltpu.async_copy(alpha_hbm, alpha_v, sem_in)
    cp_i.wait()
    cp_b.wait()

    def beta_step(i, carry):
        off = pl.ds(i * _L, _L)
        bout_v[off] = plsc.load_gather(beta_v, [idx_v[off]])
        return carry

    lax.fori_loop(0, _BPW // _L, beta_step, 0, unroll=4)
    cp_ob = pltpu.async_copy(bout_v, beta_out.at[pl.ds(base, _BPW)], sem_out)
    cp_a.wait()

    def alpha_step(i, carry):
        off = pl.ds(i * _L, _L)
        aout_v[off] = plsc.load_gather(alpha_v, [idx_v[off]])
        return carry

    lax.fori_loop(0, _BPW // _L, alpha_step, 0, unroll=4)
    cp_oa = pltpu.async_copy(aout_v, alpha_out.at[pl.ds(base, _BPW)], sem_out)
    cp_ob.wait()
    cp_oa.wait()


_ddpm = functools.partial(
    pl.kernel,
    mesh=plsc.VectorSubcoreMesh(core_axis_name="c", subcore_axis_name="s",
                                num_cores=1),
    out_type=(
        jax.ShapeDtypeStruct((_B,), jnp.float32),
        jax.ShapeDtypeStruct((_B,), jnp.float32),
    ),
    scratch_types=[
        pltpu.VMEM((_BPW,), jnp.int32),
        pltpu.VMEM((_T,), jnp.float32),
        pltpu.VMEM((_T,), jnp.float32),
        pltpu.VMEM((_BPW,), jnp.float32),
        pltpu.VMEM((_BPW,), jnp.float32),
        pltpu.SemaphoreType.DMA,
        pltpu.SemaphoreType.DMA,
    ],
    compiler_params=pltpu.CompilerParams(needs_layout_passes=False),
)(_ddpm_body)


@jax.jit
def kernel(t, beta, alpha):
    beta_t, alpha_t = _ddpm(t, beta, alpha)
    return beta_t, alpha_t


# speedup vs baseline: 1.0414x; 1.0350x over previous
import functools

import jax
import jax.numpy as jnp
from jax import lax
from jax.experimental import pallas as pl
from jax.experimental.pallas import tpu as pltpu, tpu_sc as plsc

_B = 16384
_T = 1000
_NS = 16
_L = 16
_BPW = _B // _NS


def _ddpm_body(t_hbm, beta_hbm, alpha_hbm, beta_out, alpha_out,
               idx_v, beta_v, alpha_v, bout_v, aout_v, sem_in, sem_out):
    wid = lax.axis_index("s")
    base = wid * _BPW
    cp_i = pltpu.async_copy(t_hbm.at[pl.ds(base, _BPW)], idx_v, sem_in)
    cp_b = pltpu.async_copy(beta_hbm, beta_v, sem_in)
    cp_a = pltpu.async_copy(alpha_hbm, alpha_v, sem_in)
    cp_i.wait()
    cp_b.wait()

    @plsc.parallel_loop(0, _BPW, step=_L, unroll=4)
    def _beta_pass(i):
        off = pl.ds(i, _L)
        bout_v[off] = plsc.load_gather(beta_v, [idx_v[off]])

    cp_ob = pltpu.async_copy(bout_v, beta_out.at[pl.ds(base, _BPW)], sem_out)
    cp_a.wait()

    @plsc.parallel_loop(0, _BPW, step=_L, unroll=4)
    def _alpha_pass(i):
        off = pl.ds(i, _L)
        aout_v[off] = plsc.load_gather(alpha_v, [idx_v[off]])

    cp_oa = pltpu.async_copy(aout_v, alpha_out.at[pl.ds(base, _BPW)], sem_out)
    cp_ob.wait()
    cp_oa.wait()


_ddpm = functools.partial(
    pl.kernel,
    mesh=plsc.VectorSubcoreMesh(core_axis_name="c", subcore_axis_name="s",
                                num_cores=1),
    out_type=(
        jax.ShapeDtypeStruct((_B,), jnp.float32),
        jax.ShapeDtypeStruct((_B,), jnp.float32),
    ),
    scratch_types=[
        pltpu.VMEM((_BPW,), jnp.int32),
        pltpu.VMEM((_T,), jnp.float32),
        pltpu.VMEM((_T,), jnp.float32),
        pltpu.VMEM((_BPW,), jnp.float32),
        pltpu.VMEM((_BPW,), jnp.float32),
        pltpu.SemaphoreType.DMA,
        pltpu.SemaphoreType.DMA,
    ],
    compiler_params=pltpu.CompilerParams(needs_layout_passes=False),
)(_ddpm_body)


@jax.jit
def kernel(t, beta, alpha):
    beta_t, alpha_t = _ddpm(t, beta, alpha)
    return beta_t, alpha_t
